# Initial kernel scaffold; baseline (speedup 1.0000x reference)
import jax
import jax.numpy as jnp
from jax.experimental import pallas as pl


def kernel(data, conv0, conv1, conv2, lin1, outp):
    raise NotImplementedError("write your pallas kernel here")


# speedup vs baseline: 8.9472x; 8.9472x over previous
import functools

import jax
import jax.numpy as jnp
from jax import lax
from jax.experimental import pallas as pl
from jax.experimental.pallas import tpu as pltpu
from jax.experimental.pallas import tpu_sc as plsc

F = 16
N = 1024
KNN = 20
NK = N * KNN
CNT = float(F * N * KNN)
EPS = 1e-5
RB = 256


def _knn_body(x_ref, xt_ref, idx_ref):
    f = pl.program_id(0)
    x = x_ref[0]
    xt = xt_ref[0]
    g = jnp.dot(x, xt, preferred_element_type=jnp.float32)
    sq = jnp.sum(xt * xt, axis=0, keepdims=True)
    work = sq - 2.0 * g
    col = lax.broadcasted_iota(jnp.int32, (RB, N), 1)
    kio = lax.broadcasted_iota(jnp.int32, (RB, KNN), 1)
    acc = jnp.zeros((RB, KNN), jnp.int32)
    for k in range(KNN):
        m = jnp.min(work, axis=1, keepdims=True)
        cand = jnp.where(work == m, col, jnp.int32(2**30))
        amin = jnp.min(cand, axis=1, keepdims=True)
        acc = jnp.where(kio == k, amin + f * N, acc)
        work = jnp.where(col == amin, jnp.float32(jnp.inf), work)
    idx_ref[0] = acc


def _knn(x, xt):
    dp = x.shape[2]
    return pl.pallas_call(
        _knn_body,
        grid=(F, N // RB),
        in_specs=[
            pl.BlockSpec((1, RB, dp), lambda f, r: (f, r, 0)),
            pl.BlockSpec((1, dp, N), lambda f, r: (f, 0, 0)),
        ],
        out_specs=pl.BlockSpec((1, RB, KNN), lambda f, r: (f, r, 0)),
        out_shape=jax.ShapeDtypeStruct((F, N, KNN), jnp.int32),
    )(x, xt)


def _sc_gather(table, idx, d):
    b_total = idx.shape[0]
    info = plsc.get_sparse_core_info()
    nw = info.num_cores * info.num_subcores
    b_per_w = b_total // nw
    ch = 2048
    nch = b_per_w // ch
    mesh = plsc.VectorSubcoreMesh(core_axis_name="c", subcore_axis_name="s")

    @functools.partial(
        pl.kernel, mesh=mesh,
        compiler_params=pltpu.CompilerParams(use_tc_tiling_on_sc=False),
        out_type=jax.ShapeDtypeStruct((b_total, d), jnp.float32),
        scratch_types=[
            pltpu.VMEM((ch,), jnp.int32),
            pltpu.VMEM((ch, d), jnp.float32),
            pltpu.SemaphoreType.DMA,
        ],
    )
    def k(table_hbm, idx_hbm, out_hbm, idx_v, rows_v, sem):
        wid = lax.axis_index("s") * info.num_cores + lax.axis_index("c")
        for c in range(nch):
            base = wid * b_per_w + c * ch
            pltpu.sync_copy(idx_hbm.at[pl.ds(base, ch)], idx_v)
            pltpu.async_copy(table_hbm.at[idx_v], rows_v, sem).wait()
            pltpu.sync_copy(rows_v, out_hbm.at[pl.ds(base, ch)])

    return k(table, idx)


def _acc_sums(s_ref, h, f):
    s0 = jnp.sum(h, axis=0, keepdims=True)
    s1 = jnp.sum(h * h, axis=0, keepdims=True)
    r8 = lax.broadcasted_iota(jnp.int32, (8, 32), 0)
    contrib = (jnp.where(r8 == 0, jnp.broadcast_to(s0, (8, 32)), 0.0)
               + jnp.where(r8 == 1, jnp.broadcast_to(s1, (8, 32)), 0.0))

    @pl.when(f == 0)
    def _():
        s_ref[...] = contrib

    @pl.when(f > 0)
    def _():
        s_ref[...] = s_ref[...] + contrib


def _bn_fold(s_ref, g_ref, be_ref):
    s = s_ref[...]
    mean = s[0:1, :] * (1.0 / CNT)
    var = s[1:2, :] * (1.0 / CNT) - mean * mean
    a = g_ref[...] * lax.rsqrt(var + EPS)
    c = be_ref[...] - mean * a
    return a, c


def _e1_body(x_ref, nb_ref, wt_ref, wb_ref, b_ref, h_ref, s_ref):
    f = pl.program_id(0)
    x = x_ref[0]
    dp = x.shape[1]
    xa = jnp.dot(x, wt_ref[...], preferred_element_type=jnp.float32)
    xar = jnp.reshape(
        lax.broadcast_in_dim(xa, (N, KNN, 32), (0, 2)), (NK, 32))
    xbr = jnp.reshape(
        lax.broadcast_in_dim(x, (N, KNN, dp), (0, 2)), (NK, dp))
    ed = nb_ref[0] - xbr
    h = xar + jnp.dot(ed, wb_ref[...],
                      preferred_element_type=jnp.float32) + b_ref[...]
    h_ref[0] = h
    _acc_sums(s_ref, h, f)


def _e1(x, nb, wt, wb, b):
    dp = x.shape[2]
    full = lambda f: (0, 0)
    return pl.pallas_call(
        _e1_body,
        grid=(F,),
        in_specs=[
            pl.BlockSpec((1, N, dp), lambda f: (f, 0, 0)),
            pl.BlockSpec((1, NK, dp), lambda f: (f, 0, 0)),
            pl.BlockSpec((dp, 32), full),
            pl.BlockSpec((dp, 32), full),
            pl.BlockSpec((1, 32), full),
        ],
        out_specs=[
            pl.BlockSpec((1, NK, 32), lambda f: (f, 0, 0)),
            pl.BlockSpec((8, 32), full),
        ],
        out_shape=[
            jax.ShapeDtypeStruct((F, NK, 32), jnp.float32),
            jax.ShapeDtypeStruct((8, 32), jnp.float32),
        ],
    )(x, nb, wt, wb, b)


def _e2_body(h_ref, sin_ref, g_ref, be_ref, w_ref, b_ref, ho_ref, so_ref):
    f = pl.program_id(0)
    a, c = _bn_fold(sin_ref, g_ref, be_ref)
    h = jnp.maximum(h_ref[0] * a + c, 0.0)
    ho = jnp.dot(h, w_ref[...], preferred_element_type=jnp.float32) + b_ref[...]
    ho_ref[0] = ho
    _acc_sums(so_ref, ho, f)


def _e2(h, s, g, be, w, b):
    full = lambda f: (0, 0)
    return pl.pallas_call(
        _e2_body,
        grid=(F,),
        in_specs=[
            pl.BlockSpec((1, NK, 32), lambda f: (f, 0, 0)),
            pl.BlockSpec((8, 32), full),
            pl.BlockSpec((1, 32), full),
            pl.BlockSpec((1, 32), full),
            pl.BlockSpec((32, 32), full),
            pl.BlockSpec((1, 32), full),
        ],
        out_specs=[
            pl.BlockSpec((1, NK, 32), lambda f: (f, 0, 0)),
            pl.BlockSpec((8, 32), full),
        ],
        out_shape=[
            jax.ShapeDtypeStruct((F, NK, 32), jnp.float32),
            jax.ShapeDtypeStruct((8, 32), jnp.float32),
        ],
    )(h, s, g, be, w, b)


def _e3_body(h_ref, sin_ref, g_ref, be_ref, w_ref, b_ref, xo_ref):
    a, c = _bn_fold(sin_ref, g_ref, be_ref)
    h = jnp.maximum(h_ref[0] * a + c, 0.0)
    h2 = jnp.dot(h, w_ref[...], preferred_element_type=jnp.float32) + b_ref[...]
    xo_ref[0] = jnp.max(jnp.reshape(h2, (N, KNN, 32)), axis=1)


def _e3(h, s, g, be, w, b):
    full = lambda f: (0, 0)
    return pl.pallas_call(
        _e3_body,
        grid=(F,),
        in_specs=[
            pl.BlockSpec((1, NK, 32), lambda f: (f, 0, 0)),
            pl.BlockSpec((8, 32), full),
            pl.BlockSpec((1, 32), full),
            pl.BlockSpec((1, 32), full),
            pl.BlockSpec((32, 32), full),
            pl.BlockSpec((1, 32), full),
        ],
        out_specs=pl.BlockSpec((1, N, 32), lambda f: (f, 0, 0)),
        out_shape=jax.ShapeDtypeStruct((F, N, 32), jnp.float32),
    )(h, s, g, be, w, b)


def _pool_body(x0_ref, x1_ref, x2_ref, w0_ref, w1_ref, w2_ref, b_ref, g_ref):
    h = (jnp.dot(x0_ref[0], w0_ref[...], preferred_element_type=jnp.float32)
         + jnp.dot(x1_ref[0], w1_ref[...], preferred_element_type=jnp.float32)
         + jnp.dot(x2_ref[0], w2_ref[...], preferred_element_type=jnp.float32)
         + b_ref[...])
    g_ref[0] = jnp.max(h, axis=0, keepdims=True)


def _pool(x0, x1, x2, w0, w1, w2, b):
    full = lambda f: (0, 0)
    xspec = pl.BlockSpec((1, N, 32), lambda f: (f, 0, 0))
    wspec = pl.BlockSpec((32, 1024), full)
    return pl.pallas_call(
        _pool_body,
        grid=(F,),
        in_specs=[xspec, xspec, xspec, wspec, wspec, wspec,
                  pl.BlockSpec((1, 1024), full)],
        out_specs=pl.BlockSpec((1, 1, 1024), lambda f: (f, 0, 0)),
        out_shape=jax.ShapeDtypeStruct((F, 1, 1024), jnp.float32),
    )(x0, x1, x2, w0, w1, w2, b)


def _head_body(g_ref, w0_ref, b0_ref, w1_ref, b1_ref, w2_ref, b2_ref,
               w3_ref, b3_ref, o_ref):
    h = jnp.maximum(jnp.dot(g_ref[...], w0_ref[...],
                            preferred_element_type=jnp.float32) + b0_ref[...], 0.0)
    h = jnp.maximum(jnp.dot(h, w1_ref[...],
                            preferred_element_type=jnp.float32) + b1_ref[...], 0.0)
    h = jnp.maximum(jnp.dot(h, w2_ref[...],
                            preferred_element_type=jnp.float32) + b2_ref[...], 0.0)
    o_ref[...] = jnp.dot(h, w3_ref[...],
                         preferred_element_type=jnp.float32) + b3_ref[...]


def _head(g, w0, b0, w1, b1, w2, b2, w3, b3):
    return pl.pallas_call(
        _head_body,
        out_shape=jax.ShapeDtypeStruct((F, 8), jnp.float32),
    )(g, w0, b0, w1, b1, w2, b2, w3, b3)


def kernel(data, conv0, conv1, conv2, lin1, outp):
    b_, t_, n_, c_ = data.shape
    x = data.reshape(F, N, c_)
    x = jnp.pad(x, ((0, 0), (0, 0), (0, 16 - c_)))
    feats = []
    for p, d in ((conv0, 9), (conv1, 32), (conv2, 32)):
        w0, b0, g0, be0, w1, b1, g1, be1, w2, b2 = p
        dp = x.shape[2]
        wt, wb_ = w0[:d], w0[d:]
        if dp != d:
            wt = jnp.pad(wt, ((0, dp - d), (0, 0)))
            wb_ = jnp.pad(wb_, ((0, dp - d), (0, 0)))
        xt = jnp.swapaxes(x, 1, 2)
        idx = _knn(x, xt)
        nb = _sc_gather(x.reshape(F * N, dp), idx.reshape(F * N * KNN), dp)
        nb = nb.reshape(F, NK, dp)
        h0, s0 = _e1(x, nb, wt, wb_, b0.reshape(1, 32))
        h1, s1 = _e2(h0, s0, g0.reshape(1, 32), be0.reshape(1, 32),
                     w1, b1.reshape(1, 32))
        x = _e3(h1, s1, g1.reshape(1, 32), be1.reshape(1, 32),
                w2, b2.reshape(1, 32))
        feats.append(x)
    lw, lb = lin1
    g = _pool(feats[0], feats[1], feats[2],
              lw[0:32], lw[32:64], lw[64:96], lb.reshape(1, 1024))
    g = g.reshape(F, 1024)
    w0, b0, w1, b1, w2, b2, w3, b3 = outp
    w3p = jnp.pad(w3, ((0, 0), (0, 2)))
    b3p = jnp.pad(b3, (0, 2)).reshape(1, 8)
    o = _head(g, w0, b0.reshape(1, 1024), w1, b1.reshape(1, 256),
              w2, b2.reshape(1, 128), w3p, b3p)
    return o[:, :6].reshape(b_, t_, 6)
